# baseline (device time: 48052 ns/iter reference)
import jax
import jax.numpy as jnp
from jax import lax
from jax.experimental import pallas as pl
from jax.experimental.pallas import tpu as pltpu

N_GLOBAL = 4096
EPS = 1e-5
SUB = 1024
BM = 2 * SUB
NSLOT = 4


def kernel(x, gamma):
    m, n = x.shape
    n_steps = m // BM
    sub8 = SUB // 128

    def body(x_ref, gamma_ref, out_ref, ssq_ref, recv_ref, send_sems, recv_sems):
        k = pl.program_id(0)
        my_x = lax.axis_index("x")
        my_y = lax.axis_index("y")
        nbr = (my_x, 1 - my_y)

        @pl.when(k == 0)
        def _():
            barrier_sem = pltpu.get_barrier_semaphore()
            pl.semaphore_signal(
                barrier_sem, inc=1, device_id=nbr,
                device_id_type=pl.DeviceIdType.MESH,
            )
            pl.semaphore_wait(barrier_sem, 1)

        rdmas = []
        for h in range(2):
            q = lax.rem(2 * k + h, NSLOT)
            xv = x_ref[pl.ds(h * SUB, SUB), :]
            ssq_col = jnp.sum(xv * xv, axis=1, keepdims=True)
            stacked = jnp.concatenate(
                [ssq_col[g * 128:(g + 1) * 128, :] for g in range(sub8)],
                axis=1,
            )
            ssq_ref[pl.ds(q, 1), :, :] = jnp.transpose(stacked)[None]
            rdma = pltpu.make_async_remote_copy(
                src_ref=ssq_ref.at[q],
                dst_ref=recv_ref.at[q],
                send_sem=send_sems.at[q],
                recv_sem=recv_sems.at[q],
                device_id=nbr,
                device_id_type=pl.DeviceIdType.MESH,
            )
            rdma.start()
            rdmas.append(rdma)

        for h in range(2):
            q = lax.rem(2 * k + h, NSLOT)
            rdmas[h].wait_recv()
            total8 = ssq_ref[q, :, :] + recv_ref[q, :, :]
            invT = lax.rsqrt(
                jnp.transpose(total8) * (1.0 / N_GLOBAL) + EPS
            )
            inv_col = jnp.concatenate(
                [invT[:, g:g + 1] for g in range(sub8)], axis=0
            )
            xv = x_ref[pl.ds(h * SUB, SUB), :]
            out_ref[pl.ds(h * SUB, SUB), :] = (
                xv * gamma_ref[:, :] * inv_col
            ).astype(jnp.bfloat16)

        for h in range(2):
            rdmas[h].wait_send()

    return pl.pallas_call(
        body,
        grid=(n_steps,),
        out_shape=jax.ShapeDtypeStruct((m, n), jnp.bfloat16),
        in_specs=[
            pl.BlockSpec((BM, n), lambda k: (k, 0)),
            pl.BlockSpec((1, n), lambda k: (0, 0)),
        ],
        out_specs=pl.BlockSpec((BM, n), lambda k: (k, 0)),
        scratch_shapes=[
            pltpu.VMEM((NSLOT, sub8, 128), jnp.float32),
            pltpu.VMEM((NSLOT, sub8, 128), jnp.float32),
            pltpu.SemaphoreType.DMA((NSLOT,)),
            pltpu.SemaphoreType.DMA((NSLOT,)),
        ],
        compiler_params=pltpu.CompilerParams(
            collective_id=0, vmem_limit_bytes=60 * 1024 * 1024
        ),
    )(x, gamma.reshape(1, n))
